# Initial kernel scaffold; baseline (speedup 1.0000x reference)
import jax
import jax.numpy as jnp
from jax.experimental import pallas as pl


def kernel(ph_encoding, ph2word, edge_index, etypes, ggc1_linW, ggc1_linb, ggc1_Wih, ggc1_Whh, ggc1_bih, ggc1_bhh, ggc2_linW, ggc2_linb, ggc2_Wih, ggc2_Whh, ggc2_bih, ggc2_bhh):
    raise NotImplementedError("write your pallas kernel here")


# speedup vs baseline: 58.9882x; 58.9882x over previous
import functools

import jax
import jax.numpy as jnp
from jax import lax
from jax.experimental import pallas as pl
from jax.experimental.pallas import tpu as pltpu
from jax.experimental.pallas import tpu_sc as plsc

_B = 16
_TP = 2500
_TW = 625
_H = 128
_E = 160000
_K = 6
_N = _B * _TW
_W1 = _TW + 1
_WP = 632
_KH = _K * _H

_NC = 2
_NS = 16
_NW = _NC * _NS

_SEG = _B * _W1
_RT = _B * _TP
_PCH = 128
_PNCHG = -(-_RT // _PCH)
_PNCH = -(-_PNCHG // _NW)

_E_PER = _E // _NW
_ECH = 128
_ENCH = -(-_E_PER // _ECH)

_ZB = 624
_ZS = 104
_F32 = jnp.float32


def _chunk_positions(n_per, ch):
    nch = -(-n_per // ch)
    offs = jnp.minimum(jnp.arange(nch) * ch, n_per - ch)
    pos = offs[:, None] + jnp.arange(ch)[None, :]
    dup = pos < (jnp.arange(nch) * ch)[:, None]
    return pos, dup


def _strided_chunks(n_total, ch):
    nchg = -(-n_total // ch)
    nslots = -(-nchg // _NW)
    cid = (jnp.arange(nslots)[None, :] * _NW
           + jnp.arange(_NW)[:, None])
    offs = jnp.minimum(cid * ch, n_total - ch)
    pos = offs[:, :, None] + jnp.arange(ch)[None, None, :]
    dup = pos < (cid * ch)[:, :, None]
    return pos, dup


@functools.cache
def _mesh():
    return plsc.VectorSubcoreMesh(core_axis_name="c", subcore_axis_name="s",
                                  num_cores=_NC, num_subcores=_NS)


def _pool_body(x_hbm, idx_hbm, zsum_hbm,
               hsum_hbm,
               hsum_acc, idx_v, xbuf, stg, sem):
    c = lax.axis_index("c")
    s = lax.axis_index("s")
    w = s * _NC + c
    _TL = _SEG - _NS * _ZB
    pltpu.sync_copy(idx_hbm.at[w], idx_v)
    pltpu.sync_copy(zsum_hbm, stg)
    for t in range(_ZB // _ZS):
        pltpu.sync_copy(stg, hsum_acc.at[pl.ds(s * _ZB + t * _ZS, _ZS)])

    @pl.when(s == 0)
    def _zero_tail():
        pltpu.sync_copy(stg.at[pl.ds(0, _TL)],
                        hsum_acc.at[pl.ds(_NS * _ZB, _TL)])

    plsc.subcore_barrier()

    def chunk(j, carry):
        cid = j * _NW + w
        off = pl.multiple_of(jnp.minimum(cid * _PCH, _RT - _PCH), 8)
        pltpu.async_copy(x_hbm.at[pl.ds(off, _PCH)], xbuf, sem).wait()
        pltpu.sync_copy(xbuf, hsum_acc.at[idx_v.at[j]], add=True)
        return carry

    lax.fori_loop(0, _PNCH, chunk, 0)
    plsc.subcore_barrier()
    for t in range(_ZB // _ZS):
        row = s * _ZB + t * _ZS
        pltpu.sync_copy(hsum_acc.at[pl.ds(row, _ZS)], stg)
        pltpu.sync_copy(stg, hsum_hbm.at[pl.ds(c * _SEG + row, _ZS)])

    @pl.when(s == 0)
    def _copy_tail():
        pltpu.sync_copy(hsum_acc.at[pl.ds(_NS * _ZB, _TL)], stg.at[pl.ds(0, _TL)])
        pltpu.sync_copy(stg.at[pl.ds(0, _TL)],
                        hsum_hbm.at[pl.ds(c * _SEG + _NS * _ZB, _TL)])


@functools.cache
def _pool_call():
    return pl.kernel(
        _pool_body,
        out_type=jax.ShapeDtypeStruct((2 * _SEG, _H), _F32),
        mesh=_mesh(),
        scratch_types=[
            pltpu.VMEM_SHARED((_SEG + _NS, _H), _F32),
            pltpu.VMEM((_PNCH, _PCH), jnp.int32),
            pltpu.VMEM((_PCH, _H), _F32),
            pltpu.VMEM((_ZS, _H), _F32),
            pltpu.SemaphoreType.DMA,
        ],
    )


def _edge_body(table_hbm, gidx_hbm, dst_hbm, zrow_hbm, out_hbm,
               acc, gidx_v, dst_v, rows, stg, sem):
    c = lax.axis_index("c")
    s = lax.axis_index("s")
    w = s * _NC + c
    _TL = _N - _NS * _ZB
    pltpu.sync_copy(gidx_hbm.at[w], gidx_v)
    pltpu.sync_copy(dst_hbm.at[w], dst_v)
    pltpu.sync_copy(zrow_hbm, stg)
    for t in range(_ZB // _ZS):
        pltpu.sync_copy(stg, acc.at[pl.ds(s * _ZB + t * _ZS, _ZS)])

    @pl.when(s == 0)
    def _zero_tail():
        pltpu.sync_copy(stg.at[pl.ds(0, _TL)], acc.at[pl.ds(_NS * _ZB, _TL)])

    plsc.subcore_barrier()

    def chunk(j, carry):
        pltpu.async_copy(table_hbm.at[gidx_v.at[j]], rows, sem).wait()
        pltpu.sync_copy(rows, acc.at[dst_v.at[j]], add=True)
        return carry

    lax.fori_loop(0, _ENCH, chunk, 0)
    plsc.subcore_barrier()
    for t in range(_ZB // _ZS):
        row = s * _ZB + t * _ZS
        pltpu.sync_copy(acc.at[pl.ds(row, _ZS)], stg)
        pltpu.sync_copy(stg, out_hbm.at[pl.ds(c * _N + row, _ZS)])

    @pl.when(s == 0)
    def _copy_tail():
        pltpu.sync_copy(acc.at[pl.ds(_NS * _ZB, _TL)], stg.at[pl.ds(0, _TL)])
        pltpu.sync_copy(stg.at[pl.ds(0, _TL)],
                        out_hbm.at[pl.ds(c * _N + _NS * _ZB, _TL)])


@functools.cache
def _edge_call():
    return pl.kernel(
        _edge_body,
        out_type=jax.ShapeDtypeStruct((2 * _N, _H), _F32),
        mesh=_mesh(),
        scratch_types=[
            pltpu.VMEM_SHARED((_N + _NS, _H), _F32),
            pltpu.VMEM((_ENCH, _ECH), jnp.int32),
            pltpu.VMEM((_ENCH, _ECH), jnp.int32),
            pltpu.VMEM((_ECH, _H), _F32),
            pltpu.VMEM((_ZS, _H), _F32),
            pltpu.SemaphoreType.DMA,
        ],
    )


def _out_gather_body(wp_hbm, idx_hbm, out_hbm, idx_v, buf, sem):
    c = lax.axis_index("c")
    s = lax.axis_index("s")
    w = s * _NC + c
    pltpu.sync_copy(idx_hbm.at[w], idx_v)

    def chunk(j, carry):
        pltpu.async_copy(wp_hbm.at[idx_v.at[j]], buf, sem).wait()
        cid = j * _NW + w
        off = pl.multiple_of(jnp.minimum(cid * _PCH, _RT - _PCH), 8)
        pltpu.sync_copy(buf, out_hbm.at[pl.ds(off, _PCH)])
        return carry

    lax.fori_loop(0, _PNCH, chunk, 0)


@functools.cache
def _out_gather_call():
    return pl.kernel(
        _out_gather_body,
        out_type=jax.ShapeDtypeStruct((_RT, _H), _F32),
        mesh=_mesh(),
        scratch_types=[
            pltpu.VMEM((_PNCH, _PCH), jnp.int32),
            pltpu.VMEM((_PCH, _H), _F32),
            pltpu.SemaphoreType.DMA,
        ],
    )


_GB = 2
_BPB = _B // _GB
_MR = _BPB * _W1
_OR = _BPB * _TW
_PR = _BPB * _WP

_GN = 5
_NR = _N // _GN


def _mean_body(hsum_ref, p2w_ref, wcat_ref, bcat_ref, inp_ref, hw_ref):
    hs = hsum_ref[0, 0] + hsum_ref[1, 0]
    parts = []
    for b in range(_BPB):
        ids = p2w_ref[b, :]
        wid = lax.broadcasted_iota(jnp.int32, (_TW, 1), 0) + 1
        cnt = jnp.zeros((_TW, 1), _F32)
        tc = _TP // 5
        for t in range(5):
            seg = ids[t * tc:(t + 1) * tc][None, :]
            cnt = cnt + jnp.sum((seg == wid).astype(_F32), axis=1,
                                keepdims=True)
        h_b = hs[b * _W1 + 1:b * _W1 + _W1, :]
        parts.append(h_b / jnp.maximum(cnt, 1.0))
    inp = jnp.concatenate(parts, axis=0)
    inp_ref[0] = inp
    for k in range(_K):
        hw_ref[k, 0] = (jnp.dot(inp, wcat_ref[:, k * _H:(k + 1) * _H],
                                preferred_element_type=_F32)
                        + bcat_ref[0:1, k * _H:(k + 1) * _H])


@functools.cache
def _mean_call():
    return pl.pallas_call(
        _mean_body,
        grid=(_GB,),
        in_specs=[
            pl.BlockSpec((2, 1, _MR, _H), lambda g: (0, g, 0, 0)),
            pl.BlockSpec((_BPB, _TP), lambda g: (g, 0)),
            pl.BlockSpec((_H, _KH), lambda g: (0, 0)),
            pl.BlockSpec((1, _KH), lambda g: (0, 0)),
        ],
        out_specs=[
            pl.BlockSpec((1, _OR, _H), lambda g: (g, 0, 0)),
            pl.BlockSpec((_K, 1, _OR, _H), lambda g: (0, g, 0, 0)),
        ],
        out_shape=[
            jax.ShapeDtypeStruct((_GB, _OR, _H), _F32),
            jax.ShapeDtypeStruct((_K, _GB, _OR, _H), _F32),
        ],
    )


def _gru_math(ap_ref, h_ref, wih_ref, whh_ref, bih_ref, bhh_ref):
    a = ap_ref[0, 0] + ap_ref[1, 0]
    h = h_ref[0]
    gi = jnp.dot(a, wih_ref[...], preferred_element_type=_F32) + bih_ref[...]
    gh = jnp.dot(h, whh_ref[...], preferred_element_type=_F32) + bhh_ref[...]
    r = jax.nn.sigmoid(gi[:, :_H] + gh[:, :_H])
    z = jax.nn.sigmoid(gi[:, _H:2 * _H] + gh[:, _H:2 * _H])
    n = jnp.tanh(gi[:, 2 * _H:] + r * gh[:, 2 * _H:])
    return (1.0 - z) * n + z * h


def _gru_body(ap_ref, h_ref, wih_ref, whh_ref, bih_ref, bhh_ref,
              wcat_ref, bcat_ref, hn_ref, hw_ref):
    hn = _gru_math(ap_ref, h_ref, wih_ref, whh_ref, bih_ref, bhh_ref)
    hn_ref[0] = hn
    for k in range(_K):
        hw_ref[k, 0] = (jnp.dot(hn, wcat_ref[:, k * _H:(k + 1) * _H],
                                preferred_element_type=_F32)
                        + bcat_ref[0:1, k * _H:(k + 1) * _H])


@functools.cache
def _gru_call():
    return pl.pallas_call(
        _gru_body,
        grid=(_GN,),
        in_specs=[
            pl.BlockSpec((2, 1, _NR, _H), lambda g: (0, g, 0, 0)),
            pl.BlockSpec((1, _NR, _H), lambda g: (g, 0, 0)),
            pl.BlockSpec((_H, 3 * _H), lambda g: (0, 0)),
            pl.BlockSpec((_H, 3 * _H), lambda g: (0, 0)),
            pl.BlockSpec((1, 3 * _H), lambda g: (0, 0)),
            pl.BlockSpec((1, 3 * _H), lambda g: (0, 0)),
            pl.BlockSpec((_H, _KH), lambda g: (0, 0)),
            pl.BlockSpec((1, _KH), lambda g: (0, 0)),
        ],
        out_specs=[
            pl.BlockSpec((1, _NR, _H), lambda g: (g, 0, 0)),
            pl.BlockSpec((_K, 1, _NR, _H), lambda g: (0, g, 0, 0)),
        ],
        out_shape=[
            jax.ShapeDtypeStruct((_GN, _NR, _H), _F32),
            jax.ShapeDtypeStruct((_K, _GN, _NR, _H), _F32),
        ],
    )


def _fin_body(ap_ref, h_ref, inp_ref, g1_ref,
              wih_ref, whh_ref, bih_ref, bhh_ref, wp_ref):
    hn = _gru_math(ap_ref, h_ref, wih_ref, whh_ref, bih_ref, bhh_ref)
    out = inp_ref[0] + g1_ref[0] + hn
    zr = jnp.zeros((1, _H), _F32)
    zt = jnp.zeros((_WP - _W1, _H), _F32)
    parts = []
    for b in range(_BPB):
        parts.append(zr)
        parts.append(out[b * _TW:(b + 1) * _TW, :])
        parts.append(zt)
    wp_ref[0] = jnp.concatenate(parts, axis=0)


@functools.cache
def _fin_call():
    return pl.pallas_call(
        _fin_body,
        grid=(_GB,),
        in_specs=[
            pl.BlockSpec((2, 1, _OR, _H), lambda g: (0, g, 0, 0)),
            pl.BlockSpec((1, _OR, _H), lambda g: (g, 0, 0)),
            pl.BlockSpec((1, _OR, _H), lambda g: (g, 0, 0)),
            pl.BlockSpec((1, _OR, _H), lambda g: (g, 0, 0)),
            pl.BlockSpec((_H, 3 * _H), lambda g: (0, 0)),
            pl.BlockSpec((_H, 3 * _H), lambda g: (0, 0)),
            pl.BlockSpec((1, 3 * _H), lambda g: (0, 0)),
            pl.BlockSpec((1, 3 * _H), lambda g: (0, 0)),
        ],
        out_specs=pl.BlockSpec((1, _PR, _H), lambda g: (g, 0, 0)),
        out_shape=jax.ShapeDtypeStruct((_GB, _PR, _H), _F32),
    )


def kernel(ph_encoding, ph2word, edge_index, etypes,
           ggc1_linW, ggc1_linb, ggc1_Wih, ggc1_Whh, ggc1_bih, ggc1_bhh,
           ggc2_linW, ggc2_linb, ggc2_Wih, ggc2_Whh, ggc2_bih, ggc2_bhh):
    x2d = jnp.transpose(ph_encoding, (0, 2, 1)).reshape(_RT, _H)

    p2w = ph2word.astype(jnp.int32)
    flat_seg = (jnp.arange(_B, dtype=jnp.int32)[:, None] * _W1 + p2w).reshape(-1)
    flat_wp = (jnp.arange(_B, dtype=jnp.int32)[:, None] * _WP + p2w).reshape(-1)
    ppos, pdup = _strided_chunks(_RT, _PCH)
    sw = (jnp.arange(_NW, dtype=jnp.int32) // _NC)[:, None, None]
    idxp = jnp.where(pdup, _SEG + sw, flat_seg[ppos])
    idxf = flat_wp[ppos]

    src = edge_index[0].astype(jnp.int32).reshape(_NW, _E_PER)
    dst = edge_index[1].astype(jnp.int32).reshape(_NW, _E_PER)
    et = etypes.astype(jnp.int32).reshape(_NW, _E_PER)
    epos, edup = _chunk_positions(_E_PER, _ECH)
    gidx_c = (et * _N + src)[:, epos]
    dst_c = jnp.where(edup[None], _N + sw, dst[:, epos])

    zsum = jnp.zeros((_ZS, _H), _F32)
    zrow = jnp.zeros((_ZS, _H), _F32)

    def wprep(linW, linb, Wih, Whh, bih, bhh):
        return (jnp.transpose(linW, (2, 0, 1)).reshape(_H, _KH),
                linb.reshape(1, _KH),
                Wih.T, Whh.T, bih.reshape(1, 3 * _H), bhh.reshape(1, 3 * _H))

    params = [wprep(ggc1_linW, ggc1_linb, ggc1_Wih, ggc1_Whh, ggc1_bih, ggc1_bhh),
              wprep(ggc2_linW, ggc2_linb, ggc2_Wih, ggc2_Whh, ggc2_bih, ggc2_bhh)]

    hsum_p = _pool_call()(x2d, idxp, zsum)
    inp2, hwt = _mean_call()(hsum_p.reshape(2, _GB, _MR, _H),
                             p2w.reshape(_B, _TP),
                             params[0][0], params[0][1])
    inp5 = inp2.reshape(_GN, _NR, _H)

    h = inp5
    g1 = inp5
    wp = None
    for li in range(2):
        wcat, bcat, wihT, whhT, bih, bhh = params[li]
        for st in range(5):
            ap = _edge_call()(hwt.reshape(_K * _N, _H), gidx_c, dst_c, zrow)
            if li == 1 and st == 4:
                wp = _fin_call()(ap.reshape(2, _GB, _OR, _H),
                                 h.reshape(_GB, _OR, _H),
                                 inp2, g1.reshape(_GB, _OR, _H),
                                 wihT, whhT, bih, bhh)
            else:
                nwcat, nbcat = (params[1][0], params[1][1]) if st == 4 else (wcat, bcat)
                h, hwt = _gru_call()(ap.reshape(2, _GN, _NR, _H), h,
                                     wihT, whhT, bih, bhh, nwcat, nbcat)
                if li == 0 and st == 4:
                    g1 = h

    out = _out_gather_call()(wp.reshape(_B * _WP, _H), idxf)
    return jnp.transpose(out.reshape(_B, _TP, _H), (0, 2, 1))
